# baseline (speedup 1.0000x reference)
import functools

import jax
import jax.numpy as jnp
from jax import lax
from jax.experimental import pallas as pl
from jax.experimental.pallas import tpu as pltpu
from jax.experimental.pallas import tpu_sc as plsc

_B, _S, _D = 4, 8192, 1024
_L = 16
_W = 128
_DW = _D // _W
_NC, _NS = 2, 16
_NW = _NC * _NS
_SEQ_W = _S // _NW
_C = 8
_NCH = _SEQ_W // _C
_R = _C * _DW

_mesh = plsc.VectorSubcoreMesh(core_axis_name="c", subcore_axis_name="s")


@functools.partial(
    pl.kernel,
    mesh=_mesh,
    out_type=jax.ShapeDtypeStruct((_B * _S * _DW, _W), jnp.float32),
    scratch_types=[
        pltpu.VMEM((_R, _W), jnp.float32),
        pltpu.VMEM((_B, _R, _W), jnp.float32),
    ],
)
def _sc_add(x_hbm, emb_hbm, out_hbm, ebuf, xbuf):
    wid = lax.axis_index("s") * _NC + lax.axis_index("c")
    seq0 = wid * _SEQ_W

    def chunk(k, carry):
        e0 = (seq0 + k * _C) * _DW
        pltpu.sync_copy(emb_hbm.at[pl.ds(e0, _R)], ebuf)
        for b in range(_B):
            pltpu.sync_copy(x_hbm.at[pl.ds(b * _S * _DW + e0, _R)], xbuf.at[b])

        def inner(i, c2):
            for g in range(_W // _L):
                sl = pl.ds(g * _L, _L)
                e = ebuf[i, sl]
                for b in range(_B):
                    xbuf[b, i, sl] = xbuf[b, i, sl] + e
            return c2

        lax.fori_loop(0, _R, inner, 0)

        for b in range(_B):
            pltpu.sync_copy(xbuf.at[b], out_hbm.at[pl.ds(b * _S * _DW + e0, _R)])
        return carry

    lax.fori_loop(0, _NCH, chunk, 0)


def kernel(x, emb_weight):
    b, s, d = x.shape
    x2 = x.reshape(b * s * (d // _W), _W)
    e2 = emb_weight.reshape(s * (d // _W), _W)
    out = _sc_add(x2, e2)
    return out.reshape(b, s, d)
